# baseline (speedup 1.0000x reference)
import functools

import jax
import jax.numpy as jnp
from jax import lax
from jax.experimental import pallas as pl
from jax.experimental.pallas import tpu as pltpu
from jax.experimental.pallas import tpu_sc as plsc

N = 10000
E = 320000
D = 128
HID = 128

NC = 2
NS = 16
DH = D // NC
EPT = E // NS
CH = 80
NCHUNK = EPT // CH
NROWCH = N // CH


def _mm_in_body(x_ref, w_ref, b_ref, o_ref):
    acc = jnp.dot(x_ref[...], w_ref[0], preferred_element_type=jnp.float32)
    o_ref[0] = jnp.maximum(acc + b_ref[0], 0.0)


def _mm_out_body(ah_ref, a2_ref, w1_ref, w2_ref, wo_ref, bo_ref, o_ref):
    h2 = jnp.maximum(
        jnp.dot(ah_ref[0], w1_ref[...][:DH], preferred_element_type=jnp.float32)
        + jnp.dot(ah_ref[1], w1_ref[...][DH:], preferred_element_type=jnp.float32)
        + jnp.dot(a2_ref[0], w2_ref[...][:DH], preferred_element_type=jnp.float32)
        + jnp.dot(a2_ref[1], w2_ref[...][DH:], preferred_element_type=jnp.float32),
        0.0,
    )
    o_ref[...] = jnp.dot(h2, wo_ref[...], preferred_element_type=jnp.float32) + bo_ref[...]


def _spmm_sc(src, dst, w, table):
    mesh = plsc.VectorSubcoreMesh(core_axis_name="c", subcore_axis_name="s")

    @functools.partial(
        pl.kernel,
        mesh=mesh,
        compiler_params=pltpu.CompilerParams(use_tc_tiling_on_sc=False),
        out_type=jax.ShapeDtypeStruct((NC, N, DH), jnp.float32),
        scratch_types=[
            pltpu.VMEM_SHARED((N, DH), jnp.float32),
            pltpu.VMEM((NCHUNK, CH), jnp.int32),
            pltpu.VMEM((NCHUNK, CH), jnp.int32),
            pltpu.VMEM((4, CH), jnp.float32),
            pltpu.VMEM((4, CH, DH), jnp.float32),
            [pltpu.SemaphoreType.DMA] * 4,
            [pltpu.SemaphoreType.DMA] * 4,
            [pltpu.SemaphoreType.DMA] * 4,
        ],
    )
    def spmm(src_hbm, dst_hbm, w_hbm, table_hbm, out_hbm,
             acc_sh, srcv, dstv, wv, rows, gsems, ssems, wsems):
        c = lax.axis_index("c")
        s = lax.axis_index("s")

        pltpu.sync_copy(src_hbm.at[s], srcv)
        pltpu.sync_copy(dst_hbm.at[s], dstv)

        def zrow(r, _):
            for f in range(DH // 16):
                rows[0, r, pl.ds(f * 16, 16)] = jnp.zeros((16,), jnp.float32)
            return 0
        lax.fori_loop(0, CH, zrow, 0)

        def zcopy(k, _):
            j = s + k * NS
            @pl.when(j < NROWCH)
            def _():
                pltpu.sync_copy(rows.at[0], acc_sh.at[pl.ds(j * CH, CH)])
            return 0
        lax.fori_loop(0, (NROWCH + NS - 1) // NS, zcopy, 0)
        plsc.subcore_barrier()

        lane_ids = [jnp.full((16,), jj, jnp.int32) for jj in range(16)]

        def issue_gather(j, b):
            pltpu.async_copy(w_hbm.at[s, j], wv.at[b], wsems[b])
            pltpu.async_copy(table_hbm.at[c].at[srcv.at[j]], rows.at[b], gsems[b])

        def wait_gather(j, b):
            pltpu.make_async_copy(w_hbm.at[s, j], wv.at[b], wsems[b]).wait()
            pltpu.make_async_copy(
                table_hbm.at[c].at[srcv.at[j]], rows.at[b], gsems[b]
            ).wait()

        def issue_scatter(j, b):
            pltpu.async_copy(rows.at[b], acc_sh.at[dstv.at[j]], ssems[b], add=True)

        def wait_scatter(j, b):
            pltpu.make_async_copy(
                rows.at[b], acc_sh.at[dstv.at[j]], ssems[b]
            ).wait()

        def scale(j, b):
            def grp(g, _):
                w16 = wv[b, pl.ds(g * 16, 16)]
                for jj in range(16):
                    we = lax.gather(
                        w16, lane_ids[jj][:, None],
                        lax.GatherDimensionNumbers(
                            offset_dims=(), collapsed_slice_dims=(0,),
                            start_index_map=(0,)),
                        (1,), mode=lax.GatherScatterMode.PROMISE_IN_BOUNDS)
                    e = g * 16 + jj
                    for f in range(DH // 16):
                        sl = pl.ds(f * 16, 16)
                        rows[b, e, sl] = rows[b, e, sl] * we
                return 0
            lax.fori_loop(0, CH // 16, grp, 0)

        def slot(j, b, do_issue=True):
            wait_gather(j, b)
            scale(j, b)
            pltpu.sync_copy(rows.at[b], acc_sh.at[dstv.at[j]], add=True)
            if do_issue:
                issue_gather(j + 4, b)

        for b in range(4):
            issue_gather(b, b)

        def quad(q, _):
            j0 = 4 * q
            for b in range(4):
                slot(j0 + b, b)
            return 0

        lax.fori_loop(0, NCHUNK // 4 - 1, quad, 0)
        jb = (NCHUNK // 4 - 1) * 4
        for t in range(6):
            slot(jb + t, t % 4, do_issue=(jb + t + 4 < NCHUNK))
        plsc.subcore_barrier()

        def dump(k, _):
            j = s + k * NS
            @pl.when(j < NROWCH)
            def _():
                pltpu.sync_copy(
                    acc_sh.at[pl.ds(j * CH, CH)],
                    out_hbm.at[c, pl.ds(j * CH, CH)],
                )
            return 0
        lax.fori_loop(0, (NROWCH + NS - 1) // NS, dump, 0)

    return spmm(src, dst, w, table)


def kernel(X, edge_index, edge_weight, W_in, b_in, W_mp1, W_mp2, W_out, b_out):
    src = edge_index[0].reshape(NS, NCHUNK, CH)
    dst = edge_index[1].reshape(NS, NCHUNK, CH)
    ew = edge_weight.reshape(NS, NCHUNK, CH)
    W_in2 = W_in.reshape(D, NC, DH).transpose(1, 0, 2)
    b_in2 = b_in.reshape(NC, 1, DH)
    b_out2 = b_out.reshape(1, 1)

    RB = 1000

    H1 = pl.pallas_call(
        _mm_in_body,
        grid=(N // RB, NC),
        in_specs=[
            pl.BlockSpec((RB, D), lambda i, c: (i, 0)),
            pl.BlockSpec((1, D, DH), lambda i, c: (c, 0, 0)),
            pl.BlockSpec((1, 1, DH), lambda i, c: (c, 0, 0)),
        ],
        out_specs=pl.BlockSpec((1, RB, DH), lambda i, c: (c, i, 0)),
        out_shape=jax.ShapeDtypeStruct((NC, N, DH), jnp.float32),
    )(X, W_in2, b_in2)

    AH = _spmm_sc(src, dst, ew, H1)
    A2H = _spmm_sc(src, dst, ew, AH)

    out = pl.pallas_call(
        _mm_out_body,
        grid=(N // RB,),
        in_specs=[
            pl.BlockSpec((NC, RB, DH), lambda i: (0, i, 0)),
            pl.BlockSpec((NC, RB, DH), lambda i: (0, i, 0)),
            pl.BlockSpec((HID, HID), lambda i: (0, 0)),
            pl.BlockSpec((HID, HID), lambda i: (0, 0)),
            pl.BlockSpec((HID, 1), lambda i: (0, 0)),
            pl.BlockSpec((1, 1), lambda i: (0, 0)),
        ],
        out_specs=pl.BlockSpec((RB, 1), lambda i: (i, 0)),
        out_shape=jax.ShapeDtypeStruct((N, 1), jnp.float32),
    )(AH, A2H, W_mp1, W_mp2, W_out, b_out2)

    return out


# speedup vs baseline: 2.0299x; 2.0299x over previous
import functools

import jax
import jax.numpy as jnp
from jax import lax
from jax.experimental import pallas as pl
from jax.experimental.pallas import tpu as pltpu
from jax.experimental.pallas import tpu_sc as plsc

N = 10000
E = 320000
D = 128
HID = 128

NC = 2
NS = 16
DH = D // NC
EPT = E // NS
CH = 80
NCHUNK = EPT // CH
NROWCH = N // CH


def _mm_in_body(x_ref, w_ref, b_ref, o_ref):
    acc = jnp.dot(x_ref[...], w_ref[0], preferred_element_type=jnp.float32)
    o_ref[0] = jnp.maximum(acc + b_ref[0], 0.0)


def _mm_out_body(ah_ref, a2_ref, w1_ref, w2_ref, wo_ref, bo_ref, o_ref):
    h2 = jnp.maximum(
        jnp.dot(ah_ref[0], w1_ref[...][:DH], preferred_element_type=jnp.float32)
        + jnp.dot(ah_ref[1], w1_ref[...][DH:], preferred_element_type=jnp.float32)
        + jnp.dot(a2_ref[0], w2_ref[...][:DH], preferred_element_type=jnp.float32)
        + jnp.dot(a2_ref[1], w2_ref[...][DH:], preferred_element_type=jnp.float32),
        0.0,
    )
    o_ref[...] = jnp.dot(h2, wo_ref[...], preferred_element_type=jnp.float32) + bo_ref[...]


def _spmm_sc(src, dst, w, table):
    mesh = plsc.VectorSubcoreMesh(core_axis_name="c", subcore_axis_name="s")

    @functools.partial(
        pl.kernel,
        mesh=mesh,
        compiler_params=pltpu.CompilerParams(use_tc_tiling_on_sc=False),
        out_type=jax.ShapeDtypeStruct((NC, N, DH), jnp.float32),
        scratch_types=[
            pltpu.VMEM_SHARED((N, DH), jnp.float32),
            pltpu.VMEM((NCHUNK, CH), jnp.int32),
            pltpu.VMEM((NCHUNK, CH), jnp.int32),
            pltpu.VMEM((2, CH), jnp.float32),
            pltpu.VMEM((2, CH, DH), jnp.float32),
            pltpu.VMEM((2, CH, DH), jnp.float32),
            [pltpu.SemaphoreType.DMA] * 2,
            [pltpu.SemaphoreType.DMA] * 2,
        ],
    )
    def spmm(src_hbm, dst_hbm, w_hbm, table_hbm, out_hbm,
             acc_sh, srcv, dstv, wv, rows, rowsS, gsems, wsems):
        c = lax.axis_index("c")
        s = lax.axis_index("s")

        pltpu.sync_copy(src_hbm.at[s], srcv)
        pltpu.sync_copy(dst_hbm.at[s], dstv)

        def zrow(r, _):
            for f in range(DH // 16):
                rows[0, r, pl.ds(f * 16, 16)] = jnp.zeros((16,), jnp.float32)
            return 0
        lax.fori_loop(0, CH, zrow, 0)

        def zcopy(k, _):
            j = s + k * NS
            @pl.when(j < NROWCH)
            def _():
                pltpu.sync_copy(rows.at[0], acc_sh.at[pl.ds(j * CH, CH)])
            return 0
        lax.fori_loop(0, (NROWCH + NS - 1) // NS, zcopy, 0)
        plsc.subcore_barrier()

        lane_ids = [jnp.full((16,), jj, jnp.int32) for jj in range(16)]

        def issue_gather(j, b):
            pltpu.async_copy(w_hbm.at[s, j], wv.at[b], wsems[b])
            pltpu.async_copy(table_hbm.at[c].at[srcv.at[j]], rows.at[b], gsems[b])

        def wait_gather(j, b):
            pltpu.make_async_copy(w_hbm.at[s, j], wv.at[b], wsems[b]).wait()
            pltpu.make_async_copy(
                table_hbm.at[c].at[srcv.at[j]], rows.at[b], gsems[b]
            ).wait()

        def scale(j, b):
            @plsc.parallel_loop(0, CH // 16)
            def _grp(g):
                w16 = wv[b, pl.ds(g * 16, 16)]
                for jj in range(16):
                    we = lax.gather(
                        w16, lane_ids[jj][:, None],
                        lax.GatherDimensionNumbers(
                            offset_dims=(), collapsed_slice_dims=(0,),
                            start_index_map=(0,)),
                        (1,), mode=lax.GatherScatterMode.PROMISE_IN_BOUNDS)
                    e = g * 16 + jj
                    for f in range(DH // 16):
                        sl = pl.ds(f * 16, 16)
                        rowsS[b, e, sl] = rows[b, e, sl] * we

        def slot(j, b, do_issue=True):
            wait_gather(j, b)
            scale(j, b)
            pltpu.sync_copy(rowsS.at[b], acc_sh.at[dstv.at[j]], add=True)
            if do_issue:
                issue_gather(j + 2, b)

        issue_gather(0, 0)
        issue_gather(1, 1)

        def pair(q, _):
            j0 = 2 * q
            slot(j0, 0)
            slot(j0 + 1, 1)
            return 0

        lax.fori_loop(0, NCHUNK // 2 - 1, pair, 0)
        slot(NCHUNK - 2, 0, do_issue=False)
        slot(NCHUNK - 1, 1, do_issue=False)
        plsc.subcore_barrier()

        def dump(k, _):
            j = s + k * NS
            @pl.when(j < NROWCH)
            def _():
                pltpu.sync_copy(
                    acc_sh.at[pl.ds(j * CH, CH)],
                    out_hbm.at[c, pl.ds(j * CH, CH)],
                )
            return 0
        lax.fori_loop(0, (NROWCH + NS - 1) // NS, dump, 0)

    return spmm(src, dst, w, table)


def kernel(X, edge_index, edge_weight, W_in, b_in, W_mp1, W_mp2, W_out, b_out):
    src = edge_index[0].reshape(NS, NCHUNK, CH)
    dst = edge_index[1].reshape(NS, NCHUNK, CH)
    ew = edge_weight.reshape(NS, NCHUNK, CH)
    W_in2 = W_in.reshape(D, NC, DH).transpose(1, 0, 2)
    b_in2 = b_in.reshape(NC, 1, DH)
    b_out2 = b_out.reshape(1, 1)

    RB = 1000

    H1 = pl.pallas_call(
        _mm_in_body,
        grid=(N // RB, NC),
        in_specs=[
            pl.BlockSpec((RB, D), lambda i, c: (i, 0)),
            pl.BlockSpec((1, D, DH), lambda i, c: (c, 0, 0)),
            pl.BlockSpec((1, 1, DH), lambda i, c: (c, 0, 0)),
        ],
        out_specs=pl.BlockSpec((1, RB, DH), lambda i, c: (c, i, 0)),
        out_shape=jax.ShapeDtypeStruct((NC, N, DH), jnp.float32),
    )(X, W_in2, b_in2)

    AH = _spmm_sc(src, dst, ew, H1)
    A2H = _spmm_sc(src, dst, ew, AH)

    out = pl.pallas_call(
        _mm_out_body,
        grid=(N // RB,),
        in_specs=[
            pl.BlockSpec((NC, RB, DH), lambda i: (0, i, 0)),
            pl.BlockSpec((NC, RB, DH), lambda i: (0, i, 0)),
            pl.BlockSpec((HID, HID), lambda i: (0, 0)),
            pl.BlockSpec((HID, HID), lambda i: (0, 0)),
            pl.BlockSpec((HID, 1), lambda i: (0, 0)),
            pl.BlockSpec((1, 1), lambda i: (0, 0)),
        ],
        out_specs=pl.BlockSpec((RB, 1), lambda i: (i, 0)),
        out_shape=jax.ShapeDtypeStruct((N, 1), jnp.float32),
    )(AH, A2H, W_mp1, W_mp2, W_out, b_out2)

    return out


# speedup vs baseline: 2.5204x; 1.2416x over previous
import functools

import jax
import jax.numpy as jnp
from jax import lax
from jax.experimental import pallas as pl
from jax.experimental.pallas import tpu as pltpu
from jax.experimental.pallas import tpu_sc as plsc

N = 10000
E = 320000
D = 128
HID = 128

NC = 2
NS = 16
DH = D // NC
EPT = E // NS
CH = 80
NCHUNK = EPT // CH
NROWCH = N // CH


def _mm_in_body(x_ref, w_ref, b_ref, o_ref):
    acc = jnp.dot(x_ref[...], w_ref[0], preferred_element_type=jnp.float32)
    o_ref[0] = jnp.maximum(acc + b_ref[0], 0.0)


def _mm_out_body(ah_ref, a2_ref, w1_ref, w2_ref, wo_ref, bo_ref, o_ref):
    h2 = jnp.maximum(
        jnp.dot(ah_ref[0], w1_ref[...][:DH], preferred_element_type=jnp.float32)
        + jnp.dot(ah_ref[1], w1_ref[...][DH:], preferred_element_type=jnp.float32)
        + jnp.dot(a2_ref[0], w2_ref[...][:DH], preferred_element_type=jnp.float32)
        + jnp.dot(a2_ref[1], w2_ref[...][DH:], preferred_element_type=jnp.float32),
        0.0,
    )
    o_ref[...] = jnp.dot(h2, wo_ref[...], preferred_element_type=jnp.float32) + bo_ref[...]


def _spmm_sc(src, dst, w, table):
    mesh = plsc.VectorSubcoreMesh(core_axis_name="c", subcore_axis_name="s")

    NB = 4

    @functools.partial(
        pl.kernel,
        mesh=mesh,
        compiler_params=pltpu.CompilerParams(use_tc_tiling_on_sc=False),
        out_type=jax.ShapeDtypeStruct((NC, N, DH), jnp.float32),
        scratch_types=[
            pltpu.VMEM_SHARED((N, DH), jnp.float32),
            pltpu.VMEM((NCHUNK, CH), jnp.int32),
            pltpu.VMEM((NB, CH), jnp.int32),
            pltpu.VMEM((NB, CH), jnp.float32),
            pltpu.VMEM((NB, CH, DH), jnp.float32),
            pltpu.VMEM((NB, CH, DH), jnp.float32),
            [pltpu.SemaphoreType.DMA] * NB,
            [pltpu.SemaphoreType.DMA] * NB,
            [pltpu.SemaphoreType.DMA] * NB,
        ],
    )
    def spmm(src_hbm, dst_hbm, w_hbm, table_hbm, out_hbm,
             acc_sh, srcv, dstv, wv, rows, rowsS, gsems, dsems, wsems):
        c = lax.axis_index("c")
        s = lax.axis_index("s")

        pltpu.sync_copy(src_hbm.at[s], srcv)

        def zrow(r, _):
            for f in range(DH // 16):
                rows[0, r, pl.ds(f * 16, 16)] = jnp.zeros((16,), jnp.float32)
            return 0
        lax.fori_loop(0, CH, zrow, 0)

        def zcopy(k, _):
            j = s + k * NS
            @pl.when(j < NROWCH)
            def _():
                pltpu.sync_copy(rows.at[0], acc_sh.at[pl.ds(j * CH, CH)])
            return 0
        lax.fori_loop(0, (NROWCH + NS - 1) // NS, zcopy, 0)
        plsc.subcore_barrier()

        lane_ids = [jnp.full((16,), jj, jnp.int32) for jj in range(16)]

        def issue_gather(j, b):
            pltpu.async_copy(w_hbm.at[s, j], wv.at[b], wsems[b])
            pltpu.async_copy(dst_hbm.at[s, j], dstv.at[b], dsems[b])
            pltpu.async_copy(table_hbm.at[c].at[srcv.at[j]], rows.at[b], gsems[b])

        def wait_gather(j, b):
            pltpu.make_async_copy(w_hbm.at[s, j], wv.at[b], wsems[b]).wait()
            pltpu.make_async_copy(dst_hbm.at[s, j], dstv.at[b], dsems[b]).wait()
            pltpu.make_async_copy(
                table_hbm.at[c].at[srcv.at[j]], rows.at[b], gsems[b]
            ).wait()

        def scale(j, b):
            @plsc.parallel_loop(0, CH // 16)
            def _grp(g):
                w16 = wv[b, pl.ds(g * 16, 16)]
                for jj in range(16):
                    we = lax.gather(
                        w16, lane_ids[jj][:, None],
                        lax.GatherDimensionNumbers(
                            offset_dims=(), collapsed_slice_dims=(0,),
                            start_index_map=(0,)),
                        (1,), mode=lax.GatherScatterMode.PROMISE_IN_BOUNDS)
                    e = g * 16 + jj
                    for f in range(DH // 16):
                        sl = pl.ds(f * 16, 16)
                        rowsS[b, e, sl] = rows[b, e, sl] * we

        def slot(j, b, do_issue=True):
            wait_gather(j, b)
            scale(j, b)
            pltpu.sync_copy(rowsS.at[b], acc_sh.at[dstv.at[b]], add=True)
            if do_issue:
                issue_gather(j + NB, b)

        for b in range(NB):
            issue_gather(b, b)

        def quad(q, _):
            j0 = NB * q
            for b in range(NB):
                slot(j0 + b, b)
            return 0

        NTAIL = NCHUNK - (NCHUNK // NB - 1) * NB
        lax.fori_loop(0, NCHUNK // NB - 1, quad, 0)
        jt = NCHUNK - NTAIL
        for t in range(NTAIL):
            slot(jt + t, (jt + t) % NB, do_issue=(jt + t + NB < NCHUNK))
        plsc.subcore_barrier()

        def dump(k, _):
            j = s + k * NS
            @pl.when(j < NROWCH)
            def _():
                pltpu.sync_copy(
                    acc_sh.at[pl.ds(j * CH, CH)],
                    out_hbm.at[c, pl.ds(j * CH, CH)],
                )
            return 0
        lax.fori_loop(0, (NROWCH + NS - 1) // NS, dump, 0)

    return spmm(src, dst, w, table)


def kernel(X, edge_index, edge_weight, W_in, b_in, W_mp1, W_mp2, W_out, b_out):
    src = edge_index[0].reshape(NS, NCHUNK, CH)
    dst = edge_index[1].reshape(NS, NCHUNK, CH)
    ew = edge_weight.reshape(NS, NCHUNK, CH)
    W_in2 = W_in.reshape(D, NC, DH).transpose(1, 0, 2)
    b_in2 = b_in.reshape(NC, 1, DH)
    b_out2 = b_out.reshape(1, 1)

    RB = 1000

    H1 = pl.pallas_call(
        _mm_in_body,
        grid=(N // RB, NC),
        in_specs=[
            pl.BlockSpec((RB, D), lambda i, c: (i, 0)),
            pl.BlockSpec((1, D, DH), lambda i, c: (c, 0, 0)),
            pl.BlockSpec((1, 1, DH), lambda i, c: (c, 0, 0)),
        ],
        out_specs=pl.BlockSpec((1, RB, DH), lambda i, c: (c, i, 0)),
        out_shape=jax.ShapeDtypeStruct((NC, N, DH), jnp.float32),
    )(X, W_in2, b_in2)

    AH = _spmm_sc(src, dst, ew, H1)
    A2H = _spmm_sc(src, dst, ew, AH)

    out = pl.pallas_call(
        _mm_out_body,
        grid=(N // RB,),
        in_specs=[
            pl.BlockSpec((NC, RB, DH), lambda i: (0, i, 0)),
            pl.BlockSpec((NC, RB, DH), lambda i: (0, i, 0)),
            pl.BlockSpec((HID, HID), lambda i: (0, 0)),
            pl.BlockSpec((HID, HID), lambda i: (0, 0)),
            pl.BlockSpec((HID, 1), lambda i: (0, 0)),
            pl.BlockSpec((1, 1), lambda i: (0, 0)),
        ],
        out_specs=pl.BlockSpec((RB, 1), lambda i: (i, 0)),
        out_shape=jax.ShapeDtypeStruct((N, 1), jnp.float32),
    )(AH, A2H, W_mp1, W_mp2, W_out, b_out2)

    return out


# speedup vs baseline: 2.7638x; 1.0966x over previous
import functools

import jax
import jax.numpy as jnp
from jax import lax
from jax.experimental import pallas as pl
from jax.experimental.pallas import tpu as pltpu
from jax.experimental.pallas import tpu_sc as plsc

N = 10000
E = 320000
D = 128
HID = 128

NC = 2
NS = 16
DH = D // NC
EPT = E // NS
CH = 80
NCHUNK = EPT // CH
NROWCH = N // CH


def _mm_in_body(x_ref, w_ref, b_ref, o_ref):
    acc = jnp.dot(x_ref[...], w_ref[0], preferred_element_type=jnp.float32)
    o_ref[0] = jnp.maximum(acc + b_ref[0], 0.0)


def _mm_out_body(ah_ref, a2_ref, w1_ref, w2_ref, wo_ref, bo_ref, o_ref):
    h2 = jnp.maximum(
        jnp.dot(ah_ref[0], w1_ref[...][:DH], preferred_element_type=jnp.float32)
        + jnp.dot(ah_ref[1], w1_ref[...][DH:], preferred_element_type=jnp.float32)
        + jnp.dot(a2_ref[0], w2_ref[...][:DH], preferred_element_type=jnp.float32)
        + jnp.dot(a2_ref[1], w2_ref[...][DH:], preferred_element_type=jnp.float32),
        0.0,
    )
    o_ref[...] = jnp.dot(h2, wo_ref[...], preferred_element_type=jnp.float32) + bo_ref[...]


def _spmm_sc(src, dst, w, table):
    mesh = plsc.VectorSubcoreMesh(core_axis_name="c", subcore_axis_name="s")

    NB = 4

    @functools.partial(
        pl.kernel,
        mesh=mesh,
        compiler_params=pltpu.CompilerParams(use_tc_tiling_on_sc=False),
        out_type=jax.ShapeDtypeStruct((NC, N, DH), jnp.float32),
        scratch_types=[
            pltpu.VMEM_SHARED((N, DH), jnp.float32),
            pltpu.VMEM((NCHUNK, CH), jnp.int32),
            pltpu.VMEM((NB, CH), jnp.int32),
            pltpu.VMEM((NB, CH), jnp.float32),
            pltpu.VMEM((NB, CH, DH), jnp.float32),
            pltpu.VMEM((NB, CH, DH), jnp.float32),
            [pltpu.SemaphoreType.DMA] * NB,
            [pltpu.SemaphoreType.DMA] * NB,
            [pltpu.SemaphoreType.DMA] * NB,
            [pltpu.SemaphoreType.DMA] * NB,
        ],
    )
    def spmm(src_hbm, dst_hbm, w_hbm, table_hbm, out_hbm,
             acc_sh, srcv, dstv, wv, rows, rowsS, gsems, dsems, wsems, ssems):
        c = lax.axis_index("c")
        s = lax.axis_index("s")

        pltpu.sync_copy(src_hbm.at[s], srcv)

        def zrow(r, _):
            for f in range(DH // 16):
                rows[0, r, pl.ds(f * 16, 16)] = jnp.zeros((16,), jnp.float32)
            return 0
        lax.fori_loop(0, CH, zrow, 0)

        def zcopy(k, _):
            j = s + k * NS
            @pl.when(j < NROWCH)
            def _():
                pltpu.sync_copy(rows.at[0], acc_sh.at[pl.ds(j * CH, CH)])
            return 0
        lax.fori_loop(0, (NROWCH + NS - 1) // NS, zcopy, 0)
        plsc.subcore_barrier()

        lane_ids = [jnp.full((16,), jj, jnp.int32) for jj in range(16)]

        def issue_gather(j, b):
            pltpu.async_copy(w_hbm.at[s, j], wv.at[b], wsems[b])
            pltpu.async_copy(dst_hbm.at[s, j], dstv.at[b], dsems[b])
            pltpu.async_copy(table_hbm.at[c].at[srcv.at[j]], rows.at[b], gsems[b])

        def wait_gather(j, b):
            pltpu.make_async_copy(w_hbm.at[s, j], wv.at[b], wsems[b]).wait()
            pltpu.make_async_copy(dst_hbm.at[s, j], dstv.at[b], dsems[b]).wait()
            pltpu.make_async_copy(
                table_hbm.at[c].at[srcv.at[j]], rows.at[b], gsems[b]
            ).wait()

        def scale(j, b):
            @plsc.parallel_loop(0, CH // 16)
            def _grp(g):
                w16 = wv[b, pl.ds(g * 16, 16)]
                for jj in range(16):
                    we = lax.gather(
                        w16, lane_ids[jj][:, None],
                        lax.GatherDimensionNumbers(
                            offset_dims=(), collapsed_slice_dims=(0,),
                            start_index_map=(0,)),
                        (1,), mode=lax.GatherScatterMode.PROMISE_IN_BOUNDS)
                    e = g * 16 + jj
                    for f in range(DH // 16):
                        sl = pl.ds(f * 16, 16)
                        rowsS[b, e, sl] = rows[b, e, sl] * we

        def issue_scatter(j, b):
            pltpu.async_copy(rowsS.at[b], acc_sh.at[dstv.at[b]], ssems[b],
                             add=True)

        def wait_scatter(b):
            pltpu.make_async_copy(
                rowsS.at[b], acc_sh.at[dstv.at[b]], ssems[b]
            ).wait()

        def slot(j, b, drain_issue=True, do_issue=True):
            if drain_issue:
                b2 = (b + 2) % NB
                wait_scatter(b2)
                if do_issue:
                    issue_gather(j + 2, b2)
            wait_gather(j, b)
            scale(j, b)
            issue_scatter(j, b)

        issue_gather(0, 0)
        issue_gather(1, 1)
        slot(0, 0, drain_issue=False)
        issue_gather(2, 2)
        slot(1, 1, drain_issue=False)
        issue_gather(3, 3)

        def quad(q, _):
            j0 = 2 + NB * q
            for t in range(NB):
                slot(j0 + t, (2 + t) % NB)
            return 0

        lax.fori_loop(0, (NCHUNK - 2) // NB - 1, quad, 0)
        for j in range(NCHUNK - 4, NCHUNK):
            slot(j, j % NB, do_issue=(j + 2 < NCHUNK))
        wait_scatter((NCHUNK - 2) % NB)
        wait_scatter((NCHUNK - 1) % NB)
        plsc.subcore_barrier()

        def dump(k, _):
            j = s + k * NS
            @pl.when(j < NROWCH)
            def _():
                pltpu.sync_copy(
                    acc_sh.at[pl.ds(j * CH, CH)],
                    out_hbm.at[c, pl.ds(j * CH, CH)],
                )
            return 0
        lax.fori_loop(0, (NROWCH + NS - 1) // NS, dump, 0)

    return spmm(src, dst, w, table)


def kernel(X, edge_index, edge_weight, W_in, b_in, W_mp1, W_mp2, W_out, b_out):
    src = edge_index[0].reshape(NS, NCHUNK, CH)
    dst = edge_index[1].reshape(NS, NCHUNK, CH)
    ew = edge_weight.reshape(NS, NCHUNK, CH)
    W_in2 = W_in.reshape(D, NC, DH).transpose(1, 0, 2)
    b_in2 = b_in.reshape(NC, 1, DH)
    b_out2 = b_out.reshape(1, 1)

    RB = 1000

    H1 = pl.pallas_call(
        _mm_in_body,
        grid=(N // RB, NC),
        in_specs=[
            pl.BlockSpec((RB, D), lambda i, c: (i, 0)),
            pl.BlockSpec((1, D, DH), lambda i, c: (c, 0, 0)),
            pl.BlockSpec((1, 1, DH), lambda i, c: (c, 0, 0)),
        ],
        out_specs=pl.BlockSpec((1, RB, DH), lambda i, c: (c, i, 0)),
        out_shape=jax.ShapeDtypeStruct((NC, N, DH), jnp.float32),
    )(X, W_in2, b_in2)

    AH = _spmm_sc(src, dst, ew, H1)
    A2H = _spmm_sc(src, dst, ew, AH)

    out = pl.pallas_call(
        _mm_out_body,
        grid=(N // RB,),
        in_specs=[
            pl.BlockSpec((NC, RB, DH), lambda i: (0, i, 0)),
            pl.BlockSpec((NC, RB, DH), lambda i: (0, i, 0)),
            pl.BlockSpec((HID, HID), lambda i: (0, 0)),
            pl.BlockSpec((HID, HID), lambda i: (0, 0)),
            pl.BlockSpec((HID, 1), lambda i: (0, 0)),
            pl.BlockSpec((1, 1), lambda i: (0, 0)),
        ],
        out_specs=pl.BlockSpec((RB, 1), lambda i: (i, 0)),
        out_shape=jax.ShapeDtypeStruct((N, 1), jnp.float32),
    )(AH, A2H, W_mp1, W_mp2, W_out, b_out2)

    return out


# speedup vs baseline: 2.8022x; 1.0139x over previous
import functools

import jax
import jax.numpy as jnp
from jax import lax
from jax.experimental import pallas as pl
from jax.experimental.pallas import tpu as pltpu
from jax.experimental.pallas import tpu_sc as plsc

N = 10000
E = 320000
D = 128
HID = 128

NC = 2
NS = 16
DH = D // NC
EPT = E // NS
CH = 80
NCHUNK = EPT // CH
NROWCH = N // CH


def _mm_in_body(x_ref, w_ref, b_ref, o_ref):
    acc = jnp.dot(x_ref[...], w_ref[0], preferred_element_type=jnp.float32)
    o_ref[0] = jnp.maximum(acc + b_ref[0], 0.0)


def _mm_out_body(ah_ref, a2_ref, w1_ref, w2_ref, wo_ref, bo_ref, o_ref):
    h2 = jnp.maximum(
        jnp.dot(ah_ref[0], w1_ref[...][:DH], preferred_element_type=jnp.float32)
        + jnp.dot(ah_ref[1], w1_ref[...][DH:], preferred_element_type=jnp.float32)
        + jnp.dot(a2_ref[0], w2_ref[...][:DH], preferred_element_type=jnp.float32)
        + jnp.dot(a2_ref[1], w2_ref[...][DH:], preferred_element_type=jnp.float32),
        0.0,
    )
    o_ref[...] = jnp.dot(h2, wo_ref[...], preferred_element_type=jnp.float32) + bo_ref[...]


def _spmm2_sc(src, dst, w, table):
    mesh = plsc.VectorSubcoreMesh(core_axis_name="c", subcore_axis_name="s")

    NB = 4

    @functools.partial(
        pl.kernel,
        mesh=mesh,
        compiler_params=pltpu.CompilerParams(use_tc_tiling_on_sc=False),
        out_type=(jax.ShapeDtypeStruct((NC, N, DH), jnp.float32),
                  jax.ShapeDtypeStruct((NC, N, DH), jnp.float32)),
        scratch_types=[
            pltpu.VMEM_SHARED((N, DH), jnp.float32),
            pltpu.VMEM((NCHUNK, CH), jnp.int32),
            pltpu.VMEM((NB, CH), jnp.int32),
            pltpu.VMEM((NB, CH), jnp.float32),
            pltpu.VMEM((NB, CH, DH), jnp.float32),
            pltpu.VMEM((NB, CH, DH), jnp.float32),
            [pltpu.SemaphoreType.DMA] * NB,
            [pltpu.SemaphoreType.DMA] * NB,
            [pltpu.SemaphoreType.DMA] * NB,
            [pltpu.SemaphoreType.DMA] * NB,
        ],
    )
    def spmm(src_hbm, dst_hbm, w_hbm, table_hbm, out1_hbm, out2_hbm,
             acc_sh, srcv, dstv, wv, rows, rowsS, gsems, dsems, wsems, ssems):
        c = lax.axis_index("c")
        s = lax.axis_index("s")

        pltpu.sync_copy(src_hbm.at[s], srcv)

        lane_ids = [jnp.full((16,), jj, jnp.int32) for jj in range(16)]

        def one_pass(table_h, out_h):
            def zrow(r, _):
                for f in range(DH // 16):
                    rows[0, r, pl.ds(f * 16, 16)] = jnp.zeros((16,), jnp.float32)
                return 0
            lax.fori_loop(0, CH, zrow, 0)

            def zcopy(k, _):
                j = s + k * NS
                @pl.when(j < NROWCH)
                def _():
                    pltpu.sync_copy(rows.at[0], acc_sh.at[pl.ds(j * CH, CH)])
                return 0
            lax.fori_loop(0, (NROWCH + NS - 1) // NS, zcopy, 0)
            plsc.subcore_barrier()

            def issue_gather(j, b):
                pltpu.async_copy(w_hbm.at[s, j], wv.at[b], wsems[b])
                pltpu.async_copy(dst_hbm.at[s, j], dstv.at[b], dsems[b])
                pltpu.async_copy(table_h.at[c].at[srcv.at[j]], rows.at[b],
                                 gsems[b])

            def wait_gather(j, b):
                pltpu.make_async_copy(w_hbm.at[s, j], wv.at[b], wsems[b]).wait()
                pltpu.make_async_copy(dst_hbm.at[s, j], dstv.at[b],
                                      dsems[b]).wait()
                pltpu.make_async_copy(
                    table_h.at[c].at[srcv.at[j]], rows.at[b], gsems[b]
                ).wait()

            def scale(j, b):
                @plsc.parallel_loop(0, CH // 16)
                def _grp(g):
                    w16 = wv[b, pl.ds(g * 16, 16)]
                    for jj in range(16):
                        we = lax.gather(
                            w16, lane_ids[jj][:, None],
                            lax.GatherDimensionNumbers(
                                offset_dims=(), collapsed_slice_dims=(0,),
                                start_index_map=(0,)),
                            (1,), mode=lax.GatherScatterMode.PROMISE_IN_BOUNDS)
                        e = g * 16 + jj
                        for f in range(DH // 16):
                            sl = pl.ds(f * 16, 16)
                            rowsS[b, e, sl] = rows[b, e, sl] * we

            def issue_scatter(j, b):
                pltpu.async_copy(rowsS.at[b], acc_sh.at[dstv.at[b]], ssems[b],
                                 add=True)

            def wait_scatter(b):
                pltpu.make_async_copy(
                    rowsS.at[b], acc_sh.at[dstv.at[b]], ssems[b]
                ).wait()

            def slot(j, b, drain_issue=True, do_issue=True):
                if drain_issue:
                    b2 = (b + 2) % NB
                    wait_scatter(b2)
                    if do_issue:
                        issue_gather(j + 2, b2)
                wait_gather(j, b)
                scale(j, b)
                issue_scatter(j, b)

            issue_gather(0, 0)
            issue_gather(1, 1)
            slot(0, 0, drain_issue=False)
            issue_gather(2, 2)
            slot(1, 1, drain_issue=False)
            issue_gather(3, 3)

            def quad(q, _):
                j0 = 2 + NB * q
                for t in range(NB):
                    slot(j0 + t, (2 + t) % NB)
                return 0

            lax.fori_loop(0, (NCHUNK - 2) // NB - 1, quad, 0)
            for j in range(NCHUNK - 4, NCHUNK):
                slot(j, j % NB, do_issue=(j + 2 < NCHUNK))
            wait_scatter((NCHUNK - 2) % NB)
            wait_scatter((NCHUNK - 1) % NB)
            plsc.subcore_barrier()

            def dump(k, _):
                j = s + k * NS
                @pl.when(j < NROWCH)
                def _():
                    pltpu.sync_copy(
                        acc_sh.at[pl.ds(j * CH, CH)],
                        out_h.at[c, pl.ds(j * CH, CH)],
                    )
                return 0
            lax.fori_loop(0, (NROWCH + NS - 1) // NS, dump, 0)
            plsc.subcore_barrier()

        one_pass(table_hbm, out1_hbm)
        one_pass(out1_hbm, out2_hbm)

    return spmm(src, dst, w, table)


def kernel(X, edge_index, edge_weight, W_in, b_in, W_mp1, W_mp2, W_out, b_out):
    src = edge_index[0].reshape(NS, NCHUNK, CH)
    dst = edge_index[1].reshape(NS, NCHUNK, CH)
    ew = edge_weight.reshape(NS, NCHUNK, CH)
    W_in2 = W_in.reshape(D, NC, DH).transpose(1, 0, 2)
    b_in2 = b_in.reshape(NC, 1, DH)
    b_out2 = b_out.reshape(1, 1)

    RB = 1000

    H1 = pl.pallas_call(
        _mm_in_body,
        grid=(N // RB, NC),
        in_specs=[
            pl.BlockSpec((RB, D), lambda i, c: (i, 0)),
            pl.BlockSpec((1, D, DH), lambda i, c: (c, 0, 0)),
            pl.BlockSpec((1, 1, DH), lambda i, c: (c, 0, 0)),
        ],
        out_specs=pl.BlockSpec((1, RB, DH), lambda i, c: (c, i, 0)),
        out_shape=jax.ShapeDtypeStruct((NC, N, DH), jnp.float32),
    )(X, W_in2, b_in2)

    AH, A2H = _spmm2_sc(src, dst, ew, H1)

    out = pl.pallas_call(
        _mm_out_body,
        grid=(N // RB,),
        in_specs=[
            pl.BlockSpec((NC, RB, DH), lambda i: (0, i, 0)),
            pl.BlockSpec((NC, RB, DH), lambda i: (0, i, 0)),
            pl.BlockSpec((HID, HID), lambda i: (0, 0)),
            pl.BlockSpec((HID, HID), lambda i: (0, 0)),
            pl.BlockSpec((HID, 1), lambda i: (0, 0)),
            pl.BlockSpec((1, 1), lambda i: (0, 0)),
        ],
        out_specs=pl.BlockSpec((RB, 1), lambda i: (i, 0)),
        out_shape=jax.ShapeDtypeStruct((N, 1), jnp.float32),
    )(AH, A2H, W_mp1, W_mp2, W_out, b_out2)

    return out


# speedup vs baseline: 2.8402x; 1.0135x over previous
import functools

import jax
import jax.numpy as jnp
from jax import lax
from jax.experimental import pallas as pl
from jax.experimental.pallas import tpu as pltpu
from jax.experimental.pallas import tpu_sc as plsc

N = 10000
E = 320000
D = 128
HID = 128

NC = 2
NS = 16
DH = D // NC
EPT = E // NS
CH = 80
NCHUNK = EPT // CH
NROWCH = N // CH


def _mm_in_body(x_ref, w_ref, b_ref, o_ref):
    acc = jnp.dot(x_ref[...], w_ref[0], preferred_element_type=jnp.float32)
    o_ref[0] = jnp.maximum(acc + b_ref[0], 0.0)


def _mm_out_body(ah_ref, a2_ref, w1_ref, w2_ref, wo_ref, bo_ref, o_ref):
    h2 = jnp.maximum(
        jnp.dot(ah_ref[0], w1_ref[...][:DH], preferred_element_type=jnp.float32)
        + jnp.dot(ah_ref[1], w1_ref[...][DH:], preferred_element_type=jnp.float32)
        + jnp.dot(a2_ref[0], w2_ref[...][:DH], preferred_element_type=jnp.float32)
        + jnp.dot(a2_ref[1], w2_ref[...][DH:], preferred_element_type=jnp.float32),
        0.0,
    )
    o_ref[...] = jnp.dot(h2, wo_ref[...], preferred_element_type=jnp.float32) + bo_ref[...]


def _spmm2_sc(src, dst, w, table):
    mesh = plsc.VectorSubcoreMesh(core_axis_name="c", subcore_axis_name="s")

    NB = 4

    @functools.partial(
        pl.kernel,
        mesh=mesh,
        compiler_params=pltpu.CompilerParams(use_tc_tiling_on_sc=False),
        out_type=(jax.ShapeDtypeStruct((NC, N, DH), jnp.float32),
                  jax.ShapeDtypeStruct((NC, N, DH), jnp.float32)),
        scratch_types=[
            pltpu.VMEM_SHARED((N, DH), jnp.float32),
            pltpu.VMEM((NCHUNK, CH), jnp.int32),
            pltpu.VMEM((NB, CH), jnp.int32),
            pltpu.VMEM((NB, CH), jnp.float32),
            pltpu.VMEM((NB, CH, DH), jnp.float32),
            pltpu.VMEM((NB, CH, DH), jnp.float32),
            [pltpu.SemaphoreType.DMA] * NB,
            [pltpu.SemaphoreType.DMA] * NB,
            [pltpu.SemaphoreType.DMA] * NB,
            [pltpu.SemaphoreType.DMA] * NB,
            pltpu.SemaphoreType.DMA,
        ],
    )
    def spmm(src_hbm, dst_hbm, w_hbm, table_hbm, out1_hbm, out2_hbm,
             acc_sh, srcv, dstv, wv, rows, rowsS, gsems, dsems, wsems, ssems,
             psem):
        c = lax.axis_index("c")
        s = lax.axis_index("s")

        pltpu.sync_copy(src_hbm.at[s], srcv)

        lane_ids = [jnp.full((16,), jj, jnp.int32) for jj in range(16)]

        def one_pass(table_h, out_h):
            def zrow(r, _):
                for f in range(DH // 16):
                    rows[0, r, pl.ds(f * 16, 16)] = jnp.zeros((16,), jnp.float32)
                return 0
            lax.fori_loop(0, CH, zrow, 0)

            def zcopy(k, _):
                j = s + k * NS
                @pl.when(j < NROWCH)
                def _():
                    pltpu.async_copy(rows.at[0], acc_sh.at[pl.ds(j * CH, CH)],
                                     psem)
                return 0
            lax.fori_loop(0, (NROWCH + NS - 1) // NS, zcopy, 0)

            def zdrain(k, _):
                j = s + k * NS
                @pl.when(j < NROWCH)
                def _():
                    pltpu.make_async_copy(
                        rows.at[0], acc_sh.at[pl.ds(j * CH, CH)], psem
                    ).wait()
                return 0
            lax.fori_loop(0, (NROWCH + NS - 1) // NS, zdrain, 0)
            plsc.subcore_barrier()

            def issue_gather(j, b):
                pltpu.async_copy(w_hbm.at[s, j], wv.at[b], wsems[b])
                pltpu.async_copy(dst_hbm.at[s, j], dstv.at[b], dsems[b])
                pltpu.async_copy(table_h.at[c].at[srcv.at[j]], rows.at[b],
                                 gsems[b])

            def wait_gather(j, b):
                pltpu.make_async_copy(w_hbm.at[s, j], wv.at[b], wsems[b]).wait()
                pltpu.make_async_copy(dst_hbm.at[s, j], dstv.at[b],
                                      dsems[b]).wait()
                pltpu.make_async_copy(
                    table_h.at[c].at[srcv.at[j]], rows.at[b], gsems[b]
                ).wait()

            def scale(j, b):
                @plsc.parallel_loop(0, CH // 16)
                def _grp(g):
                    w16 = wv[b, pl.ds(g * 16, 16)]
                    for jj in range(16):
                        we = lax.gather(
                            w16, lane_ids[jj][:, None],
                            lax.GatherDimensionNumbers(
                                offset_dims=(), collapsed_slice_dims=(0,),
                                start_index_map=(0,)),
                            (1,), mode=lax.GatherScatterMode.PROMISE_IN_BOUNDS)
                        e = g * 16 + jj
                        for f in range(DH // 16):
                            sl = pl.ds(f * 16, 16)
                            rowsS[b, e, sl] = rows[b, e, sl] * we

            def issue_scatter(j, b):
                pltpu.async_copy(rowsS.at[b], acc_sh.at[dstv.at[b]], ssems[b],
                                 add=True)

            def wait_scatter(b):
                pltpu.make_async_copy(
                    rowsS.at[b], acc_sh.at[dstv.at[b]], ssems[b]
                ).wait()

            def slot(j, b, drain_issue=True, do_issue=True):
                if drain_issue:
                    b2 = (b + 2) % NB
                    wait_scatter(b2)
                    if do_issue:
                        issue_gather(j + 2, b2)
                wait_gather(j, b)
                scale(j, b)
                issue_scatter(j, b)

            issue_gather(0, 0)
            issue_gather(1, 1)
            slot(0, 0, drain_issue=False)
            issue_gather(2, 2)
            slot(1, 1, drain_issue=False)
            issue_gather(3, 3)

            def quad(q, _):
                j0 = 2 + NB * q
                for t in range(NB):
                    slot(j0 + t, (2 + t) % NB)
                return 0

            lax.fori_loop(0, (NCHUNK - 2) // NB - 1, quad, 0)
            for j in range(NCHUNK - 4, NCHUNK):
                slot(j, j % NB, do_issue=(j + 2 < NCHUNK))
            wait_scatter((NCHUNK - 2) % NB)
            wait_scatter((NCHUNK - 1) % NB)
            plsc.subcore_barrier()

            def dump(k, _):
                j = s + k * NS
                @pl.when(j < NROWCH)
                def _():
                    pltpu.async_copy(
                        acc_sh.at[pl.ds(j * CH, CH)],
                        out_h.at[c, pl.ds(j * CH, CH)],
                        psem,
                    )
                return 0
            lax.fori_loop(0, (NROWCH + NS - 1) // NS, dump, 0)

            def ddrain(k, _):
                j = s + k * NS
                @pl.when(j < NROWCH)
                def _():
                    pltpu.make_async_copy(
                        acc_sh.at[pl.ds(j * CH, CH)],
                        out_h.at[c, pl.ds(j * CH, CH)],
                        psem,
                    ).wait()
                return 0
            lax.fori_loop(0, (NROWCH + NS - 1) // NS, ddrain, 0)
            plsc.subcore_barrier()

        one_pass(table_hbm, out1_hbm)
        one_pass(out1_hbm, out2_hbm)

    return spmm(src, dst, w, table)


def kernel(X, edge_index, edge_weight, W_in, b_in, W_mp1, W_mp2, W_out, b_out):
    src = edge_index[0].reshape(NS, NCHUNK, CH)
    dst = edge_index[1].reshape(NS, NCHUNK, CH)
    ew = edge_weight.reshape(NS, NCHUNK, CH)
    W_in2 = W_in.reshape(D, NC, DH).transpose(1, 0, 2)
    b_in2 = b_in.reshape(NC, 1, DH)
    b_out2 = b_out.reshape(1, 1)

    RB = 1000

    H1 = pl.pallas_call(
        _mm_in_body,
        grid=(N // RB, NC),
        in_specs=[
            pl.BlockSpec((RB, D), lambda i, c: (i, 0)),
            pl.BlockSpec((1, D, DH), lambda i, c: (c, 0, 0)),
            pl.BlockSpec((1, 1, DH), lambda i, c: (c, 0, 0)),
        ],
        out_specs=pl.BlockSpec((1, RB, DH), lambda i, c: (c, i, 0)),
        out_shape=jax.ShapeDtypeStruct((NC, N, DH), jnp.float32),
    )(X, W_in2, b_in2)

    AH, A2H = _spmm2_sc(src, dst, ew, H1)

    out = pl.pallas_call(
        _mm_out_body,
        grid=(N // RB,),
        in_specs=[
            pl.BlockSpec((NC, RB, DH), lambda i: (0, i, 0)),
            pl.BlockSpec((NC, RB, DH), lambda i: (0, i, 0)),
            pl.BlockSpec((HID, HID), lambda i: (0, 0)),
            pl.BlockSpec((HID, HID), lambda i: (0, 0)),
            pl.BlockSpec((HID, 1), lambda i: (0, 0)),
            pl.BlockSpec((1, 1), lambda i: (0, 0)),
        ],
        out_specs=pl.BlockSpec((RB, 1), lambda i: (i, 0)),
        out_shape=jax.ShapeDtypeStruct((N, 1), jnp.float32),
    )(AH, A2H, W_mp1, W_mp2, W_out, b_out2)

    return out
